# baseline (speedup 1.0000x reference)
import functools

import jax
import jax.numpy as jnp
from jax import lax
from jax.experimental import pallas as pl
from jax.experimental.pallas import tpu as pltpu
from jax.experimental.pallas import tpu_sc as plsc

VOC_SIZ = 100000
EMB_DIM = 64
NUM_POS = 2048
B, T = 64, 2048

_NC, _NS, _L = 2, 16, 16
_NW = _NC * _NS
_TPW = T // _NW


def _emb_body(idx_hbm, table_hbm, pos_hbm, emb_out, tok_out, pos_out,
              pos_v, idx_v, tok_v, emb_v, sem):
    wid = lax.axis_index("s") * _NC + lax.axis_index("c")
    t0 = wid * _TPW

    pltpu.sync_copy(pos_hbm.at[pl.ds(t0, _TPW)], pos_v)
    pltpu.sync_copy(pos_v, pos_out.at[pl.ds(t0, _TPW)])

    def body(b, carry):
        base = b * T + t0
        pltpu.sync_copy(idx_hbm.at[pl.ds(base, _TPW)], idx_v)
        pltpu.async_copy(table_hbm.at[idx_v], tok_v, sem).wait()
        pltpu.sync_copy(tok_v, tok_out.at[pl.ds(base, _TPW)])
        def add_row(r, c2):
            for j in range(EMB_DIM // _L):
                sl = pl.ds(j * _L, _L)
                emb_v[r, sl] = tok_v[r, sl] + pos_v[r, sl]
            return c2
        lax.fori_loop(0, _TPW, add_row, 0)
        pltpu.sync_copy(emb_v, emb_out.at[pl.ds(base, _TPW)])
        return carry

    lax.fori_loop(0, B, body, 0)


@jax.jit
def _emb_call(idx_flat, token_table, pos_table):
    mesh = plsc.VectorSubcoreMesh(core_axis_name="c", subcore_axis_name="s")
    out_type = (
        jax.ShapeDtypeStruct((B * T, EMB_DIM), jnp.float32),
        jax.ShapeDtypeStruct((B * T, EMB_DIM), jnp.float32),
        jax.ShapeDtypeStruct((NUM_POS, EMB_DIM), jnp.float32),
    )
    scratch = [
        pltpu.VMEM((_TPW, EMB_DIM), jnp.float32),
        pltpu.VMEM((_TPW,), jnp.int32),
        pltpu.VMEM((_TPW, EMB_DIM), jnp.float32),
        pltpu.VMEM((_TPW, EMB_DIM), jnp.float32),
        pltpu.SemaphoreType.DMA,
    ]
    fn = functools.partial(
        pl.kernel, mesh=mesh, out_type=out_type, scratch_types=scratch,
        compiler_params=pltpu.CompilerParams(use_tc_tiling_on_sc=False),
    )(_emb_body)
    return fn(idx_flat, token_table, pos_table)


def kernel(batInpTokSeq, token_table, pos_table):
    idx_flat = batInpTokSeq.reshape(B * T).astype(jnp.int32)
    emb, tok, pos = _emb_call(idx_flat, token_table, pos_table)
    return (emb.reshape(B, T, EMB_DIM), tok.reshape(B, T, EMB_DIM), pos)


# speedup vs baseline: 1.2920x; 1.2920x over previous
import functools

import jax
import jax.numpy as jnp
from jax import lax
from jax.experimental import pallas as pl
from jax.experimental.pallas import tpu as pltpu
from jax.experimental.pallas import tpu_sc as plsc

VOC_SIZ = 100000
EMB_DIM = 64
NUM_POS = 2048
B, T = 64, 2048

_NC, _NS, _L = 2, 16, 16
_NW = _NC * _NS
_TPW = T // _NW
_NBUF = 4


def _emb_body(idx_hbm, table_hbm, pos_hbm, emb_out, tok_out, pos_out,
              idx_v, pos_v, tok_bufs, emb_bufs,
              si, sp, sg, st, se):
    wid = lax.axis_index("s") * _NC + lax.axis_index("c")
    t0 = wid * _TPW

    cp_idx = pltpu.async_copy(idx_hbm.at[:, pl.ds(t0, _TPW)], idx_v, si)
    cp_pos = pltpu.async_copy(pos_hbm.at[pl.ds(t0, _TPW)], pos_v, sp)
    cp_idx.wait()

    for j in range(_NBUF):
        pltpu.async_copy(table_hbm.at[idx_v.at[j]], tok_bufs[j], sg[j])

    cp_pos.wait()
    pltpu.sync_copy(pos_v, pos_out.at[pl.ds(t0, _TPW)])

    for b in range(B):
        j = b % _NBUF
        base = b * T + t0
        pltpu.make_async_copy(
            table_hbm.at[idx_v.at[j]], tok_bufs[j], sg[j]).wait()
        if b >= _NBUF:
            pltpu.make_async_copy(
                emb_bufs[j],
                emb_out.at[pl.ds((b - _NBUF) * T + t0, _TPW)],
                se[j]).wait()

        def add_row(r, c, j=j):
            for k in range(EMB_DIM // _L):
                sl = pl.ds(k * _L, _L)
                emb_bufs[j][r, sl] = tok_bufs[j][r, sl] + pos_v[r, sl]
            return c
        lax.fori_loop(0, _TPW, add_row, 0)

        pltpu.async_copy(emb_bufs[j], emb_out.at[pl.ds(base, _TPW)], se[j])
        pltpu.async_copy(tok_bufs[j], tok_out.at[pl.ds(base, _TPW)], st[j])

        nb = b + _NBUF
        if nb < B:
            pltpu.make_async_copy(
                tok_bufs[j], tok_out.at[pl.ds(base, _TPW)], st[j]).wait()
            pltpu.async_copy(table_hbm.at[idx_v.at[nb]], tok_bufs[j], sg[j])

    for b in range(B - _NBUF, B):
        j = b % _NBUF
        base = b * T + t0
        pltpu.make_async_copy(
            tok_bufs[j], tok_out.at[pl.ds(base, _TPW)], st[j]).wait()
        pltpu.make_async_copy(
            emb_bufs[j], emb_out.at[pl.ds(base, _TPW)], se[j]).wait()


@jax.jit
def _emb_call(idx2d, token_table, pos_table):
    mesh = plsc.VectorSubcoreMesh(core_axis_name="c", subcore_axis_name="s")
    out_type = (
        jax.ShapeDtypeStruct((B * T, EMB_DIM), jnp.float32),
        jax.ShapeDtypeStruct((B * T, EMB_DIM), jnp.float32),
        jax.ShapeDtypeStruct((NUM_POS, EMB_DIM), jnp.float32),
    )
    scratch = [
        pltpu.VMEM((B, _TPW), jnp.int32),
        pltpu.VMEM((_TPW, EMB_DIM), jnp.float32),
        [pltpu.VMEM((_TPW, EMB_DIM), jnp.float32)] * _NBUF,
        [pltpu.VMEM((_TPW, EMB_DIM), jnp.float32)] * _NBUF,
        pltpu.SemaphoreType.DMA,
        pltpu.SemaphoreType.DMA,
        [pltpu.SemaphoreType.DMA] * _NBUF,
        [pltpu.SemaphoreType.DMA] * _NBUF,
        [pltpu.SemaphoreType.DMA] * _NBUF,
    ]
    fn = functools.partial(
        pl.kernel, mesh=mesh, out_type=out_type, scratch_types=scratch,
        compiler_params=pltpu.CompilerParams(use_tc_tiling_on_sc=False),
    )(_emb_body)
    return fn(idx2d, token_table, pos_table)


def kernel(batInpTokSeq, token_table, pos_table):
    idx2d = batInpTokSeq.astype(jnp.int32)
    emb, tok, pos = _emb_call(idx2d, token_table, pos_table)
    return (emb.reshape(B, T, EMB_DIM), tok.reshape(B, T, EMB_DIM), pos)


# speedup vs baseline: 2.4906x; 1.9278x over previous
import functools

import jax
import jax.numpy as jnp
from jax import lax
from jax.experimental import pallas as pl
from jax.experimental.pallas import tpu as pltpu
from jax.experimental.pallas import tpu_sc as plsc

VOC_SIZ = 100000
EMB_DIM = 64
NUM_POS = 2048
B, T = 64, 2048

_NC, _NS, _L = 2, 16, 16
_NW = _NC * _NS
_DPW = EMB_DIM // _NW
_NIB = 4
_NOB = 2
_G = T // _L


def _emb_body(idx_hbm, table_hbm, pos_hbm, emb_out, tok_out, pos_out,
              row_v, pos_v, idx_bufs, tok_bufs, emb_bufs,
              s_row, s_pos, s_idx, s_tok, s_emb):
    wid = lax.axis_index("s") * _NC + lax.axis_index("c")

    def compute(b, jj, d):
        ob = jj % _NOB

        def grp(i, c2):
            for k in range(4):
                sl = pl.ds((i * 4 + k) * _L, _L)
                iv = idx_bufs[jj][sl]
                tv = plsc.load_gather(row_v, [iv])
                tok_bufs[ob][sl] = tv
                emb_bufs[ob][sl] = tv + pos_v[sl]
            return c2
        lax.fori_loop(0, _G // 4, grp, 0)
        pltpu.async_copy(tok_bufs[ob], tok_out.at[b, d], s_tok[ob])
        pltpu.async_copy(emb_bufs[ob], emb_out.at[b, d], s_emb[ob])

    def wait_writes(b, jj, d):
        ob = jj % _NOB
        pltpu.make_async_copy(tok_bufs[ob], tok_out.at[b, d], s_tok[ob]).wait()
        pltpu.make_async_copy(emb_bufs[ob], emb_out.at[b, d], s_emb[ob]).wait()

    for u in range(_DPW):
        d = wid + u * _NW
        cp_row = pltpu.async_copy(table_hbm.at[d], row_v, s_row)
        cp_pos = pltpu.async_copy(pos_hbm.at[d], pos_v, s_pos)
        for j in range(_NIB):
            pltpu.async_copy(idx_hbm.at[j], idx_bufs[j], s_idx[j])
        cp_pos.wait()
        pltpu.sync_copy(pos_v, pos_out.at[d])
        cp_row.wait()

        for jj in range(_NIB):
            b = jj
            pltpu.make_async_copy(idx_hbm.at[b], idx_bufs[jj], s_idx[jj]).wait()
            if b >= _NOB:
                wait_writes(b - _NOB, jj, d)
            elif u > 0:
                wait_writes(B - _NOB + b, jj, d - _NW)
            compute(b, jj, d)
            pltpu.async_copy(idx_hbm.at[b + _NIB], idx_bufs[jj], s_idx[jj])

        def blk(g, c, d=d):
            for jj in range(_NIB):
                b = g * _NIB + jj
                pltpu.make_async_copy(
                    idx_hbm.at[b], idx_bufs[jj], s_idx[jj]).wait()
                wait_writes(b - _NOB, jj, d)
                compute(b, jj, d)

                @pl.when(b + _NIB < B)
                def _():
                    pltpu.async_copy(
                        idx_hbm.at[b + _NIB], idx_bufs[jj], s_idx[jj])
            return c
        lax.fori_loop(1, B // _NIB, blk, 0)

    d_last = wid + (_DPW - 1) * _NW
    for b in range(B - _NOB, B):
        wait_writes(b, b % _NIB, d_last)


@jax.jit
def _emb_call(idx2d, table_t, pos_t):
    mesh = plsc.VectorSubcoreMesh(core_axis_name="c", subcore_axis_name="s")
    out_type = (
        jax.ShapeDtypeStruct((B, EMB_DIM, T), jnp.float32),
        jax.ShapeDtypeStruct((B, EMB_DIM, T), jnp.float32),
        jax.ShapeDtypeStruct((EMB_DIM, NUM_POS), jnp.float32),
    )
    scratch = [
        pltpu.VMEM((VOC_SIZ,), jnp.float32),
        pltpu.VMEM((T,), jnp.float32),
        [pltpu.VMEM((T,), jnp.int32)] * _NIB,
        [pltpu.VMEM((T,), jnp.float32)] * _NOB,
        [pltpu.VMEM((T,), jnp.float32)] * _NOB,
        pltpu.SemaphoreType.DMA,
        pltpu.SemaphoreType.DMA,
        [pltpu.SemaphoreType.DMA] * _NIB,
        [pltpu.SemaphoreType.DMA] * _NOB,
        [pltpu.SemaphoreType.DMA] * _NOB,
    ]
    fn = functools.partial(
        pl.kernel, mesh=mesh, out_type=out_type, scratch_types=scratch,
        compiler_params=pltpu.CompilerParams(needs_layout_passes=False),
    )(_emb_body)
    return fn(idx2d, table_t, pos_t)


def kernel(batInpTokSeq, token_table, pos_table):
    idx2d = batInpTokSeq.astype(jnp.int32)
    emb, tok, pos = _emb_call(idx2d, token_table.T, pos_table.T)
    return (
        emb.transpose(0, 2, 1),
        tok.transpose(0, 2, 1),
        pos.T,
    )


# speedup vs baseline: 3.4720x; 1.3941x over previous
import functools

import jax
import jax.numpy as jnp
from jax import lax
from jax.experimental import pallas as pl
from jax.experimental.pallas import tpu as pltpu
from jax.experimental.pallas import tpu_sc as plsc

VOC_SIZ = 100000
EMB_DIM = 64
NUM_POS = 2048
B, T = 64, 2048

_NC, _NS, _L = 2, 16, 16
_NW = _NC * _NS
_DPW = EMB_DIM // _NW
_NIB = 4
_NOB = 2
_G = T // _L


def _emb_body(idx_hbm, table_hbm, pos_hbm, emb_out, tok_out, pos_out,
              row_v, pos_v, idx_bufs, tok_bufs, emb_bufs,
              s_row, s_pos, s_idx, s_tok, s_emb):
    wid = lax.axis_index("s") * _NC + lax.axis_index("c")

    def compute(b, jj, d):
        ob = jj % _NOB

        @plsc.parallel_loop(0, T, step=_L, unroll=8)
        def _grp(t):
            sl = pl.ds(t, _L)
            iv = idx_bufs[jj][sl]
            tv = plsc.load_gather(row_v, [iv])
            tok_bufs[ob][sl] = tv
            emb_bufs[ob][sl] = tv + pos_v[sl]
        pltpu.async_copy(tok_bufs[ob], tok_out.at[b, d], s_tok[ob])
        pltpu.async_copy(emb_bufs[ob], emb_out.at[b, d], s_emb[ob])

    def wait_writes(b, jj, d):
        ob = jj % _NOB
        pltpu.make_async_copy(tok_bufs[ob], tok_out.at[b, d], s_tok[ob]).wait()
        pltpu.make_async_copy(emb_bufs[ob], emb_out.at[b, d], s_emb[ob]).wait()

    for u in range(_DPW):
        d = wid + u * _NW
        cp_row = pltpu.async_copy(table_hbm.at[d], row_v, s_row)
        cp_pos = pltpu.async_copy(pos_hbm.at[d], pos_v, s_pos)
        for j in range(_NIB):
            pltpu.async_copy(idx_hbm.at[j], idx_bufs[j], s_idx[j])
        cp_pos.wait()
        pltpu.sync_copy(pos_v, pos_out.at[d])
        cp_row.wait()

        for jj in range(_NIB):
            b = jj
            pltpu.make_async_copy(idx_hbm.at[b], idx_bufs[jj], s_idx[jj]).wait()
            if b >= _NOB:
                wait_writes(b - _NOB, jj, d)
            elif u > 0:
                wait_writes(B - _NOB + b, jj, d - _NW)
            compute(b, jj, d)
            pltpu.async_copy(idx_hbm.at[b + _NIB], idx_bufs[jj], s_idx[jj])

        def blk(g, c, d=d):
            for jj in range(_NIB):
                b = g * _NIB + jj
                pltpu.make_async_copy(
                    idx_hbm.at[b], idx_bufs[jj], s_idx[jj]).wait()
                wait_writes(b - _NOB, jj, d)
                compute(b, jj, d)

                @pl.when(b + _NIB < B)
                def _():
                    pltpu.async_copy(
                        idx_hbm.at[b + _NIB], idx_bufs[jj], s_idx[jj])
            return c
        lax.fori_loop(1, B // _NIB, blk, 0)

    d_last = wid + (_DPW - 1) * _NW
    for b in range(B - _NOB, B):
        wait_writes(b, b % _NIB, d_last)


@jax.jit
def _emb_call(idx2d, table_t, pos_t):
    mesh = plsc.VectorSubcoreMesh(core_axis_name="c", subcore_axis_name="s")
    out_type = (
        jax.ShapeDtypeStruct((B, EMB_DIM, T), jnp.float32),
        jax.ShapeDtypeStruct((B, EMB_DIM, T), jnp.float32),
        jax.ShapeDtypeStruct((EMB_DIM, NUM_POS), jnp.float32),
    )
    scratch = [
        pltpu.VMEM((VOC_SIZ,), jnp.float32),
        pltpu.VMEM((T,), jnp.float32),
        [pltpu.VMEM((T,), jnp.int32)] * _NIB,
        [pltpu.VMEM((T,), jnp.float32)] * _NOB,
        [pltpu.VMEM((T,), jnp.float32)] * _NOB,
        pltpu.SemaphoreType.DMA,
        pltpu.SemaphoreType.DMA,
        [pltpu.SemaphoreType.DMA] * _NIB,
        [pltpu.SemaphoreType.DMA] * _NOB,
        [pltpu.SemaphoreType.DMA] * _NOB,
    ]
    fn = functools.partial(
        pl.kernel, mesh=mesh, out_type=out_type, scratch_types=scratch,
        compiler_params=pltpu.CompilerParams(needs_layout_passes=False),
    )(_emb_body)
    return fn(idx2d, table_t, pos_t)


def kernel(batInpTokSeq, token_table, pos_table):
    idx2d = batInpTokSeq.astype(jnp.int32)
    emb, tok, pos = _emb_call(idx2d, token_table.T, pos_table.T)
    return (
        emb.transpose(0, 2, 1),
        tok.transpose(0, 2, 1),
        pos.T,
    )
